# baseline (speedup 1.0000x reference)
import functools

import jax
import jax.numpy as jnp
from jax import lax
from jax.experimental import pallas as pl
from jax.experimental.pallas import tpu as pltpu
from jax.experimental.pallas import tpu_sc as plsc

NUM_ENTITIES = 1000000
NUM_RELATIONS = 1000
D = 64
B = 16384

NC = 2
NS = 16
NW = NC * NS
BPW = B // NW
CH = 128
NCHUNK = BPW // CH
GRP = CH // 16


def _transe_body(head_hbm, rel_hbm, tail_hbm, ent_hbm, reltab_hbm, out_hbm,
                 hidx, ridx, tidx, hidx2, ridx2, tidx2,
                 hrows, rrows, trows, score_v, sem):
    wid = lax.axis_index("s") * NC + lax.axis_index("c")
    base = wid * BPW

    pltpu.sync_copy(head_hbm.at[pl.ds(base, BPW)], hidx)
    pltpu.sync_copy(rel_hbm.at[pl.ds(base, BPW)], ridx)
    pltpu.sync_copy(tail_hbm.at[pl.ds(base, BPW)], tidx)

    def shift_body(k, carry):
        s = k * 16
        hidx2[pl.ds(s, 16)] = hidx[pl.ds(s, 16)] >> 1
        ridx2[pl.ds(s, 16)] = ridx[pl.ds(s, 16)] >> 1
        tidx2[pl.ds(s, 16)] = tidx[pl.ds(s, 16)] >> 1
        return carry

    lax.fori_loop(0, BPW // 16, shift_body, 0)

    lane = lax.broadcasted_iota(jnp.int32, (16,), 0)

    for j in range(NCHUNK):
        cph = pltpu.async_copy(
            ent_hbm.at[hidx2.at[pl.ds(j * CH, CH)]], hrows, sem)
        cpt = pltpu.async_copy(
            ent_hbm.at[tidx2.at[pl.ds(j * CH, CH)]], trows, sem)
        cpr = pltpu.async_copy(
            reltab_hbm.at[ridx2.at[pl.ds(j * CH, CH)]], rrows, sem)
        cph.wait()
        cpt.wait()
        cpr.wait()

        def group_body(g, carry, j=j):
            rows = g * 16 + lane
            offh = (hidx[pl.ds(j * CH + g * 16, 16)] & 1) * 64
            offr = (ridx[pl.ds(j * CH + g * 16, 16)] & 1) * 64
            offt = (tidx[pl.ds(j * CH + g * 16, 16)] & 1) * 64
            acc = jnp.zeros((16,), jnp.float32)
            for d in range(D):
                hv = plsc.load_gather(hrows, [rows, offh + d])
                rv = plsc.load_gather(rrows, [rows, offr + d])
                tv = plsc.load_gather(trows, [rows, offt + d])
                acc = acc + jnp.abs(hv + rv - tv)
            score_v[pl.ds(j * CH + g * 16, 16)] = acc
            return carry

        lax.fori_loop(0, GRP, group_body, 0)

    pltpu.sync_copy(score_v, out_hbm.at[pl.ds(base, BPW)])


@functools.partial(
    pl.kernel,
    mesh=plsc.VectorSubcoreMesh(core_axis_name="c", subcore_axis_name="s"),
    out_type=jax.ShapeDtypeStruct((B,), jnp.float32),
    compiler_params=pltpu.CompilerParams(needs_layout_passes=False),
    scratch_types=[
        pltpu.VMEM((BPW,), jnp.int32),
        pltpu.VMEM((BPW,), jnp.int32),
        pltpu.VMEM((BPW,), jnp.int32),
        pltpu.VMEM((BPW,), jnp.int32),
        pltpu.VMEM((BPW,), jnp.int32),
        pltpu.VMEM((BPW,), jnp.int32),
        pltpu.VMEM((CH, 2 * D), jnp.float32),
        pltpu.VMEM((CH, 2 * D), jnp.float32),
        pltpu.VMEM((CH, 2 * D), jnp.float32),
        pltpu.VMEM((BPW,), jnp.float32),
        pltpu.SemaphoreType.DMA,
    ],
)
def _transe_sc(head_hbm, rel_hbm, tail_hbm, ent_hbm, reltab_hbm, out_hbm,
               hidx, ridx, tidx, hidx2, ridx2, tidx2,
               hrows, rrows, trows, score_v, sem):
    _transe_body(head_hbm, rel_hbm, tail_hbm, ent_hbm, reltab_hbm, out_hbm,
                 hidx, ridx, tidx, hidx2, ridx2, tidx2,
                 hrows, rrows, trows, score_v, sem)


def kernel(head, relation, tail, entity_embeddings, relation_embeddings):
    def _pack(x, n):
        b = jax.lax.bitcast_convert_type(x, jnp.uint8)
        b = b.reshape(n // 2, 2 * D, 4)
        return jax.lax.bitcast_convert_type(b, jnp.float32)

    ent2 = _pack(entity_embeddings, NUM_ENTITIES)
    rel2 = _pack(relation_embeddings, NUM_RELATIONS)
    return _transe_sc(head, relation, tail, ent2, rel2)


# speedup vs baseline: 1.9001x; 1.9001x over previous
import functools

import jax
import jax.numpy as jnp
from jax import lax
from jax.experimental import pallas as pl
from jax.experimental.pallas import tpu as pltpu
from jax.experimental.pallas import tpu_sc as plsc

NUM_ENTITIES = 1000000
NUM_RELATIONS = 1000
D = 64
B = 16384

NC = 2
NS = 16
NW = NC * NS
BPW = B // NW
CH = 128
NCHUNK = BPW // CH
GRP = CH // 16


def _transe_body(head_hbm, rel_hbm, tail_hbm, ent_hbm, reltab_hbm, out_hbm,
                 hidx, ridx, tidx, hrows, rrows, trows, score_v, sem):
    wid = lax.axis_index("s") * NC + lax.axis_index("c")
    base = wid * BPW

    pltpu.sync_copy(head_hbm.at[pl.ds(base, BPW)], hidx)
    pltpu.sync_copy(rel_hbm.at[pl.ds(base, BPW)], ridx)
    pltpu.sync_copy(tail_hbm.at[pl.ds(base, BPW)], tidx)

    lane = lax.broadcasted_iota(jnp.int32, (16,), 0)

    for j in range(NCHUNK):
        cph = pltpu.async_copy(
            ent_hbm.at[hidx.at[pl.ds(j * CH, CH)]], hrows, sem)
        cpt = pltpu.async_copy(
            ent_hbm.at[tidx.at[pl.ds(j * CH, CH)]], trows, sem)
        cpr = pltpu.async_copy(
            reltab_hbm.at[ridx.at[pl.ds(j * CH, CH)]], rrows, sem)
        cph.wait()
        cpt.wait()
        cpr.wait()

        def group_body(g, carry, j=j):
            rows = g * 16 + lane
            acc = jnp.zeros((16,), jnp.float32)
            for d in range(D):
                col = jnp.full((16,), d, jnp.int32)
                hv = plsc.load_gather(hrows, [rows, col])
                rv = plsc.load_gather(rrows, [rows, col])
                tv = plsc.load_gather(trows, [rows, col])
                acc = acc + jnp.abs(hv + rv - tv)
            score_v[pl.ds(j * CH + g * 16, 16)] = acc
            return carry

        lax.fori_loop(0, GRP, group_body, 0)

    pltpu.sync_copy(score_v, out_hbm.at[pl.ds(base, BPW)])


@functools.partial(
    pl.kernel,
    mesh=plsc.VectorSubcoreMesh(core_axis_name="c", subcore_axis_name="s"),
    out_type=jax.ShapeDtypeStruct((B,), jnp.float32),
    compiler_params=pltpu.CompilerParams(needs_layout_passes=False),
    scratch_types=[
        pltpu.VMEM((BPW,), jnp.int32),
        pltpu.VMEM((BPW,), jnp.int32),
        pltpu.VMEM((BPW,), jnp.int32),
        pltpu.VMEM((CH, 2 * D), jnp.float32),
        pltpu.VMEM((CH, 2 * D), jnp.float32),
        pltpu.VMEM((CH, 2 * D), jnp.float32),
        pltpu.VMEM((BPW,), jnp.float32),
        pltpu.SemaphoreType.DMA,
    ],
)
def _transe_sc(head_hbm, rel_hbm, tail_hbm, ent_hbm, reltab_hbm, out_hbm,
               hidx, ridx, tidx, hrows, rrows, trows, score_v, sem):
    _transe_body(head_hbm, rel_hbm, tail_hbm, ent_hbm, reltab_hbm, out_hbm,
                 hidx, ridx, tidx, hrows, rrows, trows, score_v, sem)


def kernel(head, relation, tail, entity_embeddings, relation_embeddings):
    ent_p = jnp.pad(entity_embeddings, ((0, 0), (0, D)))
    rel_p = jnp.pad(relation_embeddings, ((0, 0), (0, D)))
    return _transe_sc(head, relation, tail, ent_p, rel_p)
